# baseline (speedup 1.0000x reference)
import jax
import jax.numpy as jnp
from jax.experimental import pallas as pl
from jax.experimental.pallas import tpu as pltpu

_BM = 1024
_BN = 512


def _mm_kernel(x_ref, w_ref, o_ref, xb_ref):
    @pl.when(pl.program_id(1) == 0)
    def _():
        xb_ref[...] = x_ref[...].astype(jnp.bfloat16)

    wb = w_ref[...].astype(jnp.bfloat16)
    o_ref[...] = jax.lax.dot_general(
        xb_ref[...], wb, (((1,), (1,)), ((), ())),
        preferred_element_type=jnp.float32)


def kernel(x, weight):
    M, K = x.shape
    N, _ = weight.shape
    return pl.pallas_call(
        _mm_kernel,
        grid=(M // _BM, N // _BN),
        in_specs=[
            pl.BlockSpec((_BM, K), lambda i, j: (i, 0)),
            pl.BlockSpec((_BN, K), lambda i, j: (j, 0)),
        ],
        out_specs=pl.BlockSpec((_BM, _BN), lambda i, j: (i, j)),
        out_shape=jax.ShapeDtypeStruct((M, N), jnp.float32),
        scratch_shapes=[pltpu.VMEM((_BM, K), jnp.bfloat16)],
    )(x, weight)
